# baseline (speedup 1.0000x reference)
import functools

import jax
import jax.numpy as jnp
from jax import lax
from jax.experimental import pallas as pl
from jax.experimental.pallas import tpu as pltpu
from jax.experimental.pallas import tpu_sc as plsc


def _gather_sc(ptab2, mtab_t, pidx, midx, n_tokens, p_dim, m_dim,
               m_vocab, n_b, s_len):
    info = plsc.get_sparse_core_info()
    nw = info.num_cores * info.num_subcores
    n_w = n_tokens // nw
    ch = 256
    n_ch = n_w // ch
    mesh = plsc.VectorSubcoreMesh(core_axis_name="c", subcore_axis_name="s")

    @functools.partial(
        pl.kernel,
        mesh=mesh,
        compiler_params=pltpu.CompilerParams(use_tc_tiling_on_sc=True,
                                             needs_layout_passes=False),
        out_type=[
            jax.ShapeDtypeStruct((n_tokens, p_dim), jnp.float32),
            jax.ShapeDtypeStruct((n_b, m_dim, s_len), jnp.float32),
        ],
        scratch_types=[
            pltpu.VMEM(ptab2.shape, jnp.float32),
            pltpu.VMEM((m_dim, m_vocab), jnp.float32),
            pltpu.VMEM((n_w,), jnp.int32),
            pltpu.VMEM((n_w,), jnp.int32),
            pltpu.VMEM((2, ch, p_dim), jnp.float32),
            pltpu.VMEM((2, m_dim, ch), jnp.float32),
            pltpu.SemaphoreType.DMA,
            pltpu.SemaphoreType.DMA,
            pltpu.SemaphoreType.DMA,
            pltpu.SemaphoreType.DMA,
        ],
    )
    def k(ptab2_h, mtab_h, pidx_h, midx_h, pout, mout,
          ptab_v, mtab_v, pidx_v, midx_v, pstage, mstage,
          sem_t, sem_op, sem_om, sem_g):
        wid = lax.axis_index("s") * info.num_cores + lax.axis_index("c")
        base = wid * n_w
        bb = base // s_len
        s0 = base % s_len
        ct = pltpu.async_copy(ptab2_h, ptab_v, sem_t)
        cm = pltpu.async_copy(mtab_h, mtab_v, sem_t)
        ci = pltpu.async_copy(pidx_h.at[pl.ds(base, n_w)], pidx_v, sem_t)
        cj = pltpu.async_copy(midx_h.at[pl.ds(base, n_w)], midx_v, sem_t)
        ct.wait()
        cm.wait()
        ci.wait()
        cj.wait()

        o = {}
        g_stream = {}
        for c in range(n_ch):
            b = c & 1
            use_stream = (c & 1) == 1
            if c >= 2:
                o[c - 2][0].wait()
                o[c - 2][1].wait()
            if use_stream:
                g_stream[c] = pltpu.async_copy(
                    ptab2_h.at[pidx_v.at[pl.ds(c * ch, ch)]],
                    pstage.at[b], sem_g)

            @plsc.parallel_loop(0, ch // 16)
            def _body(g, c=c, b=b, use_stream=use_stream):
                midxv = midx_v[pl.ds(c * ch + g * 16, 16)]
                if not use_stream:
                    pidxv = pidx_v[pl.ds(c * ch + g * 16, 16)]
                    for l in range(0, 16, 2):
                        loads = []
                        for t in (l, l + 1):
                            pi = pidxv[t]
                            loads.append([ptab_v[pi, pl.ds(kk * 16, 16)]
                                          for kk in range(p_dim // 16)])
                        for t in (l, l + 1):
                            for kk in range(p_dim // 16):
                                pstage[b, g * 16 + t, pl.ds(kk * 16, 16)] = (
                                    loads[t - l][kk])
                for d0 in range(0, m_dim, 8):
                    vals = [plsc.load_gather(mtab_v.at[d], [midxv])
                            for d in range(d0, d0 + 8)]
                    for i, d in enumerate(range(d0, d0 + 8)):
                        mstage[b, d, pl.ds(g * 16, 16)] = vals[i]

            if use_stream:
                g_stream[c].wait()
            off = base + c * ch
            o[c] = (
                pltpu.async_copy(pstage.at[b],
                                 pout.at[pl.ds(off, ch)],
                                 sem_op),
                pltpu.async_copy(mstage.at[b],
                                 mout.at[bb, :, pl.ds(s0 + c * ch, ch)],
                                 sem_om),
            )
        for c in (n_ch - 2, n_ch - 1):
            o[c][0].wait()
            o[c][1].wait()

    return k(ptab2, mtab_t, pidx, midx)


def _proj_tc(f0_bs, unv_bs, wf_col, bf_col, wu_col, bu_col):
    f0_dim = wf_col.shape[0]
    unv_dim = wu_col.shape[0]
    n_b, s_len = f0_bs.shape
    sblk = 256
    grid = (s_len // sblk,)

    def body(f0_ref, unv_ref, wf_ref, bf_ref, wu_ref, bu_ref, fo_ref, uo_ref):
        f0r = f0_ref[...]
        unr = unv_ref[...]
        fo_ref[...] = (wf_ref[...][None, :, :] * f0r[:, None, :]
                       + bf_ref[...][None, :, :])
        uo_ref[...] = (wu_ref[...][None, :, :] * unr[:, None, :]
                       + bu_ref[...][None, :, :])

    return pl.pallas_call(
        body,
        grid=grid,
        in_specs=[
            pl.BlockSpec((n_b, sblk), lambda j: (0, j)),
            pl.BlockSpec((n_b, sblk), lambda j: (0, j)),
            pl.BlockSpec((f0_dim, 1), lambda j: (0, 0)),
            pl.BlockSpec((f0_dim, 1), lambda j: (0, 0)),
            pl.BlockSpec((unv_dim, 1), lambda j: (0, 0)),
            pl.BlockSpec((unv_dim, 1), lambda j: (0, 0)),
        ],
        out_specs=[
            pl.BlockSpec((n_b, f0_dim, sblk), lambda j: (0, 0, j)),
            pl.BlockSpec((n_b, unv_dim, sblk), lambda j: (0, 0, j)),
        ],
        out_shape=[
            jax.ShapeDtypeStruct((n_b, f0_dim, s_len), jnp.float32),
            jax.ShapeDtypeStruct((n_b, unv_dim, s_len), jnp.float32),
        ],
    )(f0_bs, unv_bs, wf_col, bf_col, wu_col, bu_col)


def kernel(f0, phone_label, phone_duration, midi_label, unvoiced_flag,
           W_f0, b_f0, phone_table, midi_table, W_unv, b_unv):
    b, s = phone_label.shape
    n = b * s
    f0_dim = W_f0.shape[0]
    unv_dim = W_unv.shape[0]
    p_dim = phone_table.shape[1]
    m_dim = midi_table.shape[1]
    m_vocab = midi_table.shape[0]

    pidx = phone_label.astype(jnp.int32).reshape(n)
    midx = midi_label.astype(jnp.int32).reshape(n)
    pout, mout3 = _gather_sc(
        phone_table, midi_table.T,
        pidx, midx, n, p_dim, m_dim, m_vocab, b, s)

    fo3, uo3 = _proj_tc(
        f0.reshape(b, s), unvoiced_flag.reshape(b, s),
        W_f0, b_f0.reshape(f0_dim, 1),
        W_unv, b_unv.reshape(unv_dim, 1),
    )
    return (
        jnp.swapaxes(fo3, 1, 2),
        pout.reshape(b, s, p_dim),
        jnp.swapaxes(mout3, 1, 2),
        jnp.swapaxes(uo3, 1, 2),
    )


# speedup vs baseline: 1.4181x; 1.4181x over previous
import functools

import jax
import jax.numpy as jnp
from jax import lax
from jax.experimental import pallas as pl
from jax.experimental.pallas import tpu as pltpu
from jax.experimental.pallas import tpu_sc as plsc


def _gather_sc(ptab_flat, mtab_t, pidx, midx, n_tokens, p_dim, m_dim,
               m_vocab, n_b, s_len):
    p_words = ptab_flat.shape[0]
    info = plsc.get_sparse_core_info()
    nw = info.num_cores * info.num_subcores
    n_w = n_tokens // nw
    ch = 256
    n_ch = n_w // ch
    mesh = plsc.VectorSubcoreMesh(core_axis_name="c", subcore_axis_name="s")

    @functools.partial(
        pl.kernel,
        mesh=mesh,
        compiler_params=pltpu.CompilerParams(use_tc_tiling_on_sc=True,
                                             needs_layout_passes=False),
        out_type=[
            jax.ShapeDtypeStruct((n_tokens * p_dim,), jnp.float32),
            jax.ShapeDtypeStruct((n_b, m_dim, s_len), jnp.float32),
        ],
        scratch_types=[
            pltpu.VMEM((p_words,), jnp.float32),
            pltpu.VMEM((m_dim, m_vocab), jnp.float32),
            pltpu.VMEM((n_w,), jnp.int32),
            pltpu.VMEM((n_w,), jnp.int32),
            pltpu.VMEM((2 * ch * p_dim,), jnp.float32),
            pltpu.VMEM((2, m_dim, ch), jnp.float32),
            pltpu.SemaphoreType.DMA,
            pltpu.SemaphoreType.DMA,
            pltpu.SemaphoreType.DMA,
        ],
    )
    def k(ptab_h, mtab_h, pidx_h, midx_h, pout, mout,
          ptab_v, mtab_v, pidx_v, midx_v, pstage, mstage,
          sem_t, sem_op, sem_om):
        wid = lax.axis_index("s") * info.num_cores + lax.axis_index("c")
        base = wid * n_w
        bb = base // s_len
        s0 = base % s_len
        ct = pltpu.async_copy(ptab_h, ptab_v, sem_t)
        cm = pltpu.async_copy(mtab_h, mtab_v, sem_t)
        ci = pltpu.async_copy(pidx_h.at[pl.ds(base, n_w)], pidx_v, sem_t)
        cj = pltpu.async_copy(midx_h.at[pl.ds(base, n_w)], midx_v, sem_t)
        ct.wait()
        cm.wait()
        ci.wait()
        cj.wait()

        o = {}
        for c in range(n_ch):
            b = c & 1
            if c >= 2:
                o[c - 2][0].wait()
                o[c - 2][1].wait()

            pbase = b * ch * p_dim

            @plsc.parallel_loop(0, ch // 16)
            def _body(g, c=c, b=b, pbase=pbase):
                pidxv = pidx_v[pl.ds(c * ch + g * 16, 16)] * p_dim
                midxv = midx_v[pl.ds(c * ch + g * 16, 16)]
                for l in range(0, 16, 2):
                    loads = []
                    for t in (l, l + 1):
                        pi = pidxv[t]
                        loads.append([ptab_v[pl.ds(pi + kk * 16, 16)]
                                      for kk in range(p_dim // 16)])
                    for t in (l, l + 1):
                        sb = pbase + (g * 16 + t) * p_dim
                        for kk in range(p_dim // 16):
                            pstage[pl.ds(sb + kk * 16, 16)] = loads[t - l][kk]
                for d0 in range(0, m_dim, 8):
                    vals = [plsc.load_gather(mtab_v.at[d], [midxv])
                            for d in range(d0, d0 + 8)]
                    for i, d in enumerate(range(d0, d0 + 8)):
                        mstage[b, d, pl.ds(g * 16, 16)] = vals[i]

            off = base + c * ch
            o[c] = (
                pltpu.async_copy(pstage.at[pl.ds(pbase, ch * p_dim)],
                                 pout.at[pl.ds(off * p_dim, ch * p_dim)],
                                 sem_op),
                pltpu.async_copy(mstage.at[b],
                                 mout.at[bb, :, pl.ds(s0 + c * ch, ch)],
                                 sem_om),
            )
        for c in (n_ch - 2, n_ch - 1):
            o[c][0].wait()
            o[c][1].wait()

    return k(ptab_flat, mtab_t, pidx, midx)


def _proj_tc(f0_bs, unv_bs, wf_col, bf_col, wu_col, bu_col):
    f0_dim = wf_col.shape[0]
    unv_dim = wu_col.shape[0]
    n_b, s_len = f0_bs.shape
    sblk = 256
    grid = (s_len // sblk,)

    def body(f0_ref, unv_ref, wf_ref, bf_ref, wu_ref, bu_ref, fo_ref, uo_ref):
        f0r = f0_ref[...]
        unr = unv_ref[...]
        fo_ref[...] = (wf_ref[...][None, :, :] * f0r[:, None, :]
                       + bf_ref[...][None, :, :])
        uo_ref[...] = (wu_ref[...][None, :, :] * unr[:, None, :]
                       + bu_ref[...][None, :, :])

    return pl.pallas_call(
        body,
        grid=grid,
        in_specs=[
            pl.BlockSpec((n_b, sblk), lambda j: (0, j)),
            pl.BlockSpec((n_b, sblk), lambda j: (0, j)),
            pl.BlockSpec((f0_dim, 1), lambda j: (0, 0)),
            pl.BlockSpec((f0_dim, 1), lambda j: (0, 0)),
            pl.BlockSpec((unv_dim, 1), lambda j: (0, 0)),
            pl.BlockSpec((unv_dim, 1), lambda j: (0, 0)),
        ],
        out_specs=[
            pl.BlockSpec((n_b, f0_dim, sblk), lambda j: (0, 0, j)),
            pl.BlockSpec((n_b, unv_dim, sblk), lambda j: (0, 0, j)),
        ],
        out_shape=[
            jax.ShapeDtypeStruct((n_b, f0_dim, s_len), jnp.float32),
            jax.ShapeDtypeStruct((n_b, unv_dim, s_len), jnp.float32),
        ],
    )(f0_bs, unv_bs, wf_col, bf_col, wu_col, bu_col)


def kernel(f0, phone_label, phone_duration, midi_label, unvoiced_flag,
           W_f0, b_f0, phone_table, midi_table, W_unv, b_unv):
    b, s = phone_label.shape
    n = b * s
    f0_dim = W_f0.shape[0]
    unv_dim = W_unv.shape[0]
    p_dim = phone_table.shape[1]
    m_dim = midi_table.shape[1]
    m_vocab = midi_table.shape[0]

    pidx = phone_label.astype(jnp.int32).reshape(n)
    midx = midi_label.astype(jnp.int32).reshape(n)
    pout, mout3 = _gather_sc(
        phone_table.reshape(-1), midi_table.T,
        pidx, midx, n, p_dim, m_dim, m_vocab, b, s)

    fo3, uo3 = _proj_tc(
        f0.reshape(b, s), unvoiced_flag.reshape(b, s),
        W_f0, b_f0.reshape(f0_dim, 1),
        W_unv, b_unv.reshape(unv_dim, 1),
    )
    return (
        jnp.swapaxes(fo3, 1, 2),
        pout.reshape(b, s, p_dim),
        jnp.swapaxes(mout3, 1, 2),
        jnp.swapaxes(uo3, 1, 2),
    )
